# Initial kernel scaffold; baseline (speedup 1.0000x reference)
import jax
import jax.numpy as jnp
from jax.experimental import pallas as pl


def kernel(outputs, labels, global_logit):
    raise NotImplementedError("write your pallas kernel here")


# speedup vs baseline: 30.9344x; 30.9344x over previous
import jax
import jax.numpy as jnp
from jax.experimental import pallas as pl
from jax.experimental.pallas import tpu as pltpu

_N = 16384
_D = 128
_K = 128
_T = 2048
_NT = _N // _T
_IMAX = 0x7FFFFFFF
_INFBITS = 0x7F800000


def _body(o_ref, lab_ref, g_ref, out_ref, key_ref, nll_ref):
    i = pl.program_id(0)
    o = o_ref[...]
    g = g_ref[...]
    ot = o.T
    g_sq = jnp.sum(g * g, axis=1, keepdims=True)
    dot = jax.lax.dot_general(g, ot, (((1,), (0,)), ((), ())),
                              preferred_element_type=jnp.float32)
    dpart = g_sq - 2.0 * dot
    minv = jnp.min(dpart, axis=0, keepdims=True)
    kiota = jax.lax.broadcasted_iota(jnp.int32, dpart.shape, 0)
    amin = jnp.min(jnp.where(dpart == minv, kiota, _IMAX),
                   axis=0, keepdims=True)
    o_sq = jnp.sum(ot * ot, axis=0, keepdims=True)
    key_f = jnp.maximum(minv + o_sq, 0.0)
    m = jnp.max(ot, axis=0, keepdims=True)
    lse = m + jnp.log(jnp.sum(jnp.exp(ot - m), axis=0, keepdims=True))
    lab = lab_ref[0]
    is_u = lab > (_K - 1)
    lab_eff = jnp.where(is_u, amin, lab)
    o_at = jnp.sum(jnp.where(kiota == lab_eff, ot, 0.0),
                   axis=0, keepdims=True)
    nll = lse - o_at
    key_i = jnp.where(is_u, jax.lax.bitcast_convert_type(key_f, jnp.int32),
                      _IMAX)
    key_ref[pl.ds(i, 1), :] = key_i
    nll_ref[pl.ds(i, 1), :] = nll

    @pl.when(i == _NT - 1)
    def _finalize():
        keys = key_ref[...]
        nlls = nll_ref[...]
        num_u = jnp.sum((keys != _IMAX).astype(jnp.int32))
        num_p = jnp.int32(_N) - num_u
        num_sel = num_u // 10
        p_sum = jnp.sum(jnp.where(keys == _IMAX, nlls, 0.0))

        def bs_val(_, lohi):
            lo, hi = lohi
            mid = lo + (hi - lo) // 2
            cnt = jnp.sum((keys <= mid).astype(jnp.int32))
            ge = cnt >= num_sel
            return (jnp.where(ge, lo, mid + 1), jnp.where(ge, mid, hi))

        t, _hi = jax.lax.fori_loop(0, 31, bs_val,
                                   (jnp.int32(0), jnp.int32(_INFBITS + 1)))
        cnt_less = jnp.sum((keys < t).astype(jnp.int32))
        rem = num_sel - cnt_less

        ridx = jax.lax.broadcasted_iota(jnp.int32, keys.shape, 0)
        cidx = jax.lax.broadcasted_iota(jnp.int32, keys.shape, 1)
        idx = ridx * _T + cidx
        eq = keys == t

        def bs_idx(_, lohi):
            lo, hi = lohi
            mid = lo + (hi - lo) // 2
            cnt = jnp.sum((eq & (idx < mid)).astype(jnp.int32))
            ge = cnt >= rem
            return (jnp.where(ge, lo, mid + 1), jnp.where(ge, mid, hi))

        j, _hi2 = jax.lax.fori_loop(0, 15, bs_idx,
                                    (jnp.int32(0), jnp.int32(_N)))
        sel = (keys < t) | (eq & (idx < j))
        s_sum = jnp.sum(jnp.where(sel, nlls, 0.0))
        total = (num_p + num_sel).astype(jnp.float32)
        out_ref[0, 0] = (p_sum + s_sum) / total


def kernel(outputs, labels, global_logit):
    outputs = outputs.astype(jnp.float32)
    labels = labels.astype(jnp.int32).reshape(_NT, 1, _T)
    loss = pl.pallas_call(
        _body,
        grid=(_NT,),
        in_specs=[
            pl.BlockSpec((_T, _D), lambda i: (i, 0)),
            pl.BlockSpec((1, 1, _T), lambda i: (i, 0, 0)),
            pl.BlockSpec((_K, _D), lambda i: (0, 0)),
        ],
        out_specs=pl.BlockSpec((1, 1), lambda i: (0, 0),
                               memory_space=pltpu.SMEM),
        out_shape=jax.ShapeDtypeStruct((1, 1), jnp.float32),
        scratch_shapes=[
            pltpu.VMEM((_NT, _T), jnp.int32),
            pltpu.VMEM((_NT, _T), jnp.float32),
        ],
    )(outputs, labels, global_logit)
    return loss[0, 0]
